# baseline (speedup 1.0000x reference)
import functools

import jax
import jax.numpy as jnp
from jax import lax
from jax.experimental import pallas as pl
from jax.experimental.pallas import tpu as pltpu
from jax.experimental.pallas import tpu_sc as plsc


def _pq_body(x_ref, ws_ref, wd_ref, p_ref, q_ref):
    xb = x_ref[...]
    p_ref[...] = jnp.dot(xb, ws_ref[...], preferred_element_type=jnp.float32)
    q_ref[...] = jnp.dot(xb, wd_ref[...], preferred_element_type=jnp.float32)


def _pq(x, ws, wd):
    n, d = x.shape
    de = ws.shape[1]
    bn = 2000
    grid = (n // bn,)
    return pl.pallas_call(
        _pq_body,
        grid=grid,
        in_specs=[
            pl.BlockSpec((bn, d), lambda i: (i, 0)),
            pl.BlockSpec((d, de), lambda i: (0, 0)),
            pl.BlockSpec((d, de), lambda i: (0, 0)),
        ],
        out_specs=[
            pl.BlockSpec((bn, de), lambda i: (i, 0)),
            pl.BlockSpec((bn, de), lambda i: (i, 0)),
        ],
        out_shape=[
            jax.ShapeDtypeStruct((n, de), jnp.float32),
            jax.ShapeDtypeStruct((n, de), jnp.float32),
        ],
    )(x, ws, wd)


def _r_body(e_ref, w_ref, b_ref, r_ref):
    r_ref[...] = (
        jnp.dot(e_ref[...], w_ref[...], preferred_element_type=jnp.float32)
        + b_ref[...]
    )


def _r(e_p, w, b):
    ne8, lanes = e_p.shape
    de = w.shape[0]
    w_bd = jnp.kron(jnp.eye(lanes // de, dtype=jnp.float32), w)
    b_t = jnp.tile(b, lanes // de).reshape(1, lanes)
    be = 2000
    grid = (ne8 // be,)
    return pl.pallas_call(
        _r_body,
        grid=grid,
        in_specs=[
            pl.BlockSpec((be, lanes), lambda i: (i, 0)),
            pl.BlockSpec((lanes, lanes), lambda i: (0, 0)),
            pl.BlockSpec((1, lanes), lambda i: (0, 0)),
        ],
        out_specs=pl.BlockSpec((be, lanes), lambda i: (i, 0)),
        out_shape=jax.ShapeDtypeStruct((ne8, lanes), jnp.float32),
    )(e_p, w_bd, b_t)


def _node_body(residual, x_ref, a0_ref, a1_ref, wx_ref, wa_ref, b_ref,
               ws_ref, wd_ref, x_out, p_out, q_out):
    xb = x_ref[...]
    agg = a0_ref[...] + a1_ref[...]
    h = (
        jnp.dot(xb, wx_ref[...], preferred_element_type=jnp.float32)
        + jnp.dot(agg, wa_ref[...], preferred_element_type=jnp.float32)
        + b_ref[...]
    )
    h = jnp.maximum(h, 0.0)
    if residual:
        h = h + xb
    x_out[...] = h
    p_out[...] = jnp.dot(h, ws_ref[...], preferred_element_type=jnp.float32)
    q_out[...] = jnp.dot(h, wd_ref[...], preferred_element_type=jnp.float32)


def _node(x, a0, a1, wx, wa, b, ws_next, wd_next, residual):
    n, d = x.shape
    de = a0.shape[1]
    bn = 2000
    grid = (n // bn,)
    return pl.pallas_call(
        functools.partial(_node_body, residual),
        grid=grid,
        in_specs=[
            pl.BlockSpec((bn, d), lambda i: (i, 0)),
            pl.BlockSpec((bn, de), lambda i: (i, 0)),
            pl.BlockSpec((bn, de), lambda i: (i, 0)),
            pl.BlockSpec((d, d), lambda i: (0, 0)),
            pl.BlockSpec((de, d), lambda i: (0, 0)),
            pl.BlockSpec((1, d), lambda i: (0, 0)),
            pl.BlockSpec((d, de), lambda i: (0, 0)),
            pl.BlockSpec((d, de), lambda i: (0, 0)),
        ],
        out_specs=[
            pl.BlockSpec((bn, d), lambda i: (i, 0)),
            pl.BlockSpec((bn, de), lambda i: (i, 0)),
            pl.BlockSpec((bn, de), lambda i: (i, 0)),
        ],
        out_shape=[
            jax.ShapeDtypeStruct((n, d), jnp.float32),
            jax.ShapeDtypeStruct((n, de), jnp.float32),
            jax.ShapeDtypeStruct((n, de), jnp.float32),
        ],
    )(x, a0, a1, wx, wa, b.reshape(1, d), ws_next, wd_next)


_CK = 512
_SUB = 128


def _edge_sc_call(src, dst, p, q, r, prev, with_agg):
    chunks, nsub, sub = src.shape
    e_edges = chunks * nsub * sub
    n, de = p.shape
    residual = prev is not None

    try:
        info = plsc.get_sparse_core_info()
        nc, ns = int(info.num_cores), int(info.num_subcores)
    except Exception:
        nc, ns = 2, 16
    nw = nc * ns

    maxj = (chunks + nw - 1) // nw
    rows = ((n + ns - 1) // ns + 7) // 8 * 8
    n_pad = rows * ns

    out_type = [jax.ShapeDtypeStruct((e_edges, de), jnp.float32)]
    if with_agg:
        out_type.append(jax.ShapeDtypeStruct((nc * n_pad, de), jnp.float32))

    scratch = [
        pltpu.VMEM((nsub, _SUB), jnp.int32),
        pltpu.VMEM((nsub, _SUB), jnp.int32),
        pltpu.VMEM((_CK, de), jnp.float32),
        pltpu.VMEM((_CK, de), jnp.float32),
        pltpu.VMEM((_CK, de), jnp.float32),
        pltpu.VMEM((_CK, de), jnp.float32),
        pltpu.VMEM_SHARED((n, de), jnp.float32),
        pltpu.VMEM_SHARED((n, de), jnp.float32),
    ]
    if residual:
        scratch.append(pltpu.VMEM((_CK, de), jnp.float32))
        scratch.append(pltpu.VMEM((_CK, de), jnp.float32))
    if with_agg:
        scratch.append(pltpu.VMEM_SHARED((n_pad, de), jnp.float32))
        scratch.append(pltpu.VMEM((rows, de), jnp.float32))

    mesh = plsc.VectorSubcoreMesh(core_axis_name="c", subcore_axis_name="s")

    def body(*refs):
        idx = 5 + residual
        src_h, dst_h, p_h, q_h, r_h = refs[:5]
        prev_h = refs[5] if residual else None
        e_out = refs[idx]
        agg_out = refs[idx + 1] if with_agg else None
        sidx, didx, p_b, q_b, r_b, e_b, p_s, q_s = refs[
            idx + 1 + with_agg:idx + 9 + with_agg]
        off = idx + 9 + with_agg
        if residual:
            prev_b, o_b = refs[off], refs[off + 1]
            off += 2
        if with_agg:
            agg_s, z_b = refs[off], refs[off + 1]

        c = lax.axis_index("c")
        s = lax.axis_index("s")
        w = s * nc + c

        @pl.when(s == 0)
        def _():
            pltpu.sync_copy(p_h, p_s)

        @pl.when(s == 1)
        def _():
            pltpu.sync_copy(q_h, q_s)

        if with_agg:
            def zrow(i, carry):
                z_b[i] = jnp.zeros((de,), jnp.float32)
                return carry
            lax.fori_loop(0, rows, zrow, 0)
            pltpu.sync_copy(z_b, agg_s.at[pl.ds(s * rows, rows)])
        plsc.subcore_barrier()

        def chunk_body(j, carry):
            chunk = w + j * nw

            @pl.when(chunk < chunks)
            def _():
                base = chunk * _CK
                pltpu.sync_copy(src_h.at[chunk], sidx)
                pltpu.sync_copy(dst_h.at[chunk], didx)
                pltpu.sync_copy(r_h.at[pl.ds(base, _CK)], r_b)
                if residual:
                    pltpu.sync_copy(prev_h.at[pl.ds(base, _CK)], prev_b)
                for k in range(nsub):
                    pltpu.sync_copy(p_s.at[sidx.at[k]],
                                    p_b.at[pl.ds(k * _SUB, _SUB)])
                    pltpu.sync_copy(q_s.at[didx.at[k]],
                                    q_b.at[pl.ds(k * _SUB, _SUB)])

                def row(i, carry):
                    e = jnp.maximum(p_b[i] + q_b[i] + r_b[i], 0.0)
                    e_b[i] = e
                    if residual:
                        o_b[i] = prev_b[i] + e
                    return carry
                lax.fori_loop(0, _CK, row, 0)

                if with_agg:
                    for k in range(nsub):
                        pltpu.sync_copy(e_b.at[pl.ds(k * _SUB, _SUB)],
                                        agg_s.at[didx.at[k]], add=True)
                pltpu.sync_copy(o_b if residual else e_b,
                                e_out.at[pl.ds(base, _CK)])
            return carry

        lax.fori_loop(0, maxj, chunk_body, 0)

        if with_agg:
            plsc.subcore_barrier()
            pltpu.sync_copy(agg_s.at[pl.ds(s * rows, rows)],
                            agg_out.at[pl.ds(c * n_pad + s * rows, rows)])

    fn = pl.kernel(
        body,
        out_type=out_type,
        scratch_types=scratch,
        mesh=mesh,
        compiler_params=pltpu.CompilerParams(use_tc_tiling_on_sc=False),
    )
    args = (src, dst, p, q, r) + ((prev,) if residual else ())
    outs = fn(*args)
    if with_agg:
        return outs[0], outs[1]
    return outs[0], None


def kernel(edge_index, x, z, We0, be0, Wn0, bn0, We1, be1, Wn1, bn1,
           We2, be2, Wn2, bn2):
    n, d = x.shape
    de = z.shape[1]
    n_edges = edge_index.shape[1]
    src = edge_index[0].reshape(n_edges // _CK, _CK // _SUB, _SUB)
    dst = edge_index[1].reshape(n_edges // _CK, _CK // _SUB, _SUB)
    x = x.astype(jnp.float32)
    n_pad = ((n + 15) // 16 + 7) // 8 * 8 * 16

    p0, q0 = _pq(x, We0[:d], We0[d:2 * d])
    r0 = _r(z, We0[2 * d:], be0)
    e0, agg0 = _edge_sc_call(src, dst, p0, q0, r0, prev=None, with_agg=True)
    x1, p1, q1 = _node(x, agg0[:n], agg0[n_pad:n_pad + n], Wn0[:d], Wn0[d:],
                       bn0, We1[:d], We1[d:2 * d], residual=False)

    r1 = _r(e0, We1[2 * d:], be1)
    e1, agg1 = _edge_sc_call(src, dst, p1, q1, r1, prev=e0, with_agg=True)
    x2, p2, q2 = _node(x1, agg1[:n], agg1[n_pad:n_pad + n], Wn1[:d], Wn1[d:],
                       bn1, We2[:d], We2[d:2 * d], residual=True)

    r2 = _r(e1, We2[2 * d:], be2)
    e2, _ = _edge_sc_call(src, dst, p2, q2, r2, prev=None, with_agg=False)
    return e2


# speedup vs baseline: 2.1967x; 2.1967x over previous
import functools

import jax
import jax.numpy as jnp
from jax import lax
from jax.experimental import pallas as pl
from jax.experimental.pallas import tpu as pltpu
from jax.experimental.pallas import tpu_sc as plsc


def _pq_body(x_ref, ws_ref, wd_ref, p_ref, q_ref):
    xb = x_ref[...]
    p_ref[...] = jnp.dot(xb, ws_ref[...], preferred_element_type=jnp.float32)
    q_ref[...] = jnp.dot(xb, wd_ref[...], preferred_element_type=jnp.float32)


def _pq(x, ws, wd):
    n, d = x.shape
    de = ws.shape[1]
    bn = 2000
    grid = (n // bn,)
    return pl.pallas_call(
        _pq_body,
        grid=grid,
        in_specs=[
            pl.BlockSpec((bn, d), lambda i: (i, 0)),
            pl.BlockSpec((d, de), lambda i: (0, 0)),
            pl.BlockSpec((d, de), lambda i: (0, 0)),
        ],
        out_specs=[
            pl.BlockSpec((bn, de), lambda i: (i, 0)),
            pl.BlockSpec((bn, de), lambda i: (i, 0)),
        ],
        out_shape=[
            jax.ShapeDtypeStruct((n, de), jnp.float32),
            jax.ShapeDtypeStruct((n, de), jnp.float32),
        ],
    )(x, ws, wd)


def _r_body(e_ref, w_ref, b_ref, r_ref):
    r_ref[...] = (
        jnp.dot(e_ref[...], w_ref[...], preferred_element_type=jnp.float32)
        + b_ref[...]
    )


def _r(e_p, w, b):
    ne8, lanes = e_p.shape
    de = w.shape[0]
    w_bd = jnp.kron(jnp.eye(lanes // de, dtype=jnp.float32), w)
    b_t = jnp.tile(b, lanes // de).reshape(1, lanes)
    be = 2000
    grid = (ne8 // be,)
    return pl.pallas_call(
        _r_body,
        grid=grid,
        in_specs=[
            pl.BlockSpec((be, lanes), lambda i: (i, 0)),
            pl.BlockSpec((lanes, lanes), lambda i: (0, 0)),
            pl.BlockSpec((1, lanes), lambda i: (0, 0)),
        ],
        out_specs=pl.BlockSpec((be, lanes), lambda i: (i, 0)),
        out_shape=jax.ShapeDtypeStruct((ne8, lanes), jnp.float32),
    )(e_p, w_bd, b_t)


def _node_body(residual, x_ref, a0_ref, a1_ref, wx_ref, wa_ref, b_ref,
               ws_ref, wd_ref, x_out, p_out, q_out):
    xb = x_ref[...]
    agg = a0_ref[...] + a1_ref[...]
    h = (
        jnp.dot(xb, wx_ref[...], preferred_element_type=jnp.float32)
        + jnp.dot(agg, wa_ref[...], preferred_element_type=jnp.float32)
        + b_ref[...]
    )
    h = jnp.maximum(h, 0.0)
    if residual:
        h = h + xb
    x_out[...] = h
    p_out[...] = jnp.dot(h, ws_ref[...], preferred_element_type=jnp.float32)
    q_out[...] = jnp.dot(h, wd_ref[...], preferred_element_type=jnp.float32)


def _node(x, a0, a1, wx, wa, b, ws_next, wd_next, residual):
    n, d = x.shape
    de = a0.shape[1]
    bn = 2000
    grid = (n // bn,)
    return pl.pallas_call(
        functools.partial(_node_body, residual),
        grid=grid,
        in_specs=[
            pl.BlockSpec((bn, d), lambda i: (i, 0)),
            pl.BlockSpec((bn, de), lambda i: (i, 0)),
            pl.BlockSpec((bn, de), lambda i: (i, 0)),
            pl.BlockSpec((d, d), lambda i: (0, 0)),
            pl.BlockSpec((de, d), lambda i: (0, 0)),
            pl.BlockSpec((1, d), lambda i: (0, 0)),
            pl.BlockSpec((d, de), lambda i: (0, 0)),
            pl.BlockSpec((d, de), lambda i: (0, 0)),
        ],
        out_specs=[
            pl.BlockSpec((bn, d), lambda i: (i, 0)),
            pl.BlockSpec((bn, de), lambda i: (i, 0)),
            pl.BlockSpec((bn, de), lambda i: (i, 0)),
        ],
        out_shape=[
            jax.ShapeDtypeStruct((n, d), jnp.float32),
            jax.ShapeDtypeStruct((n, de), jnp.float32),
            jax.ShapeDtypeStruct((n, de), jnp.float32),
        ],
    )(x, a0, a1, wx, wa, b.reshape(1, d), ws_next, wd_next)


_CK = 512
_SUB = 128


def _edge_sc_call(src, dst, p, q, r, prev, with_agg):
    chunks, nsub, sub = src.shape
    e_edges = chunks * nsub * sub
    n, de = p.shape
    pk = _CK // 8
    residual = prev is not None

    try:
        info = plsc.get_sparse_core_info()
        nc, ns = int(info.num_cores), int(info.num_subcores)
    except Exception:
        nc, ns = 2, 16
    nw = nc * ns

    maxj = (chunks + nw - 1) // nw
    rows = ((n + ns - 1) // ns + 7) // 8 * 8
    n_pad = rows * ns

    out_type = [jax.ShapeDtypeStruct((e_edges // 8, 8 * de), jnp.float32)]
    if with_agg:
        out_type.append(jax.ShapeDtypeStruct((nc * n_pad, de), jnp.float32))

    scratch = [
        pltpu.VMEM((nsub, _SUB), jnp.int32),
        pltpu.VMEM((nsub, _SUB), jnp.int32),
        pltpu.VMEM((_CK, de), jnp.float32),
        pltpu.VMEM((_CK, de), jnp.float32),
        pltpu.VMEM((pk, 8 * de), jnp.float32),
        pltpu.VMEM((pk, 8 * de), jnp.float32),
        pltpu.VMEM_SHARED((n, de), jnp.float32),
        pltpu.VMEM_SHARED((n, de), jnp.float32),
    ]
    if residual:
        scratch.append(pltpu.VMEM((pk, 8 * de), jnp.float32))
    if with_agg:
        scratch.append(pltpu.VMEM((_CK, de), jnp.float32))
        scratch.append(pltpu.VMEM_SHARED((n_pad, de), jnp.float32))
        scratch.append(pltpu.VMEM((rows, de), jnp.float32))

    mesh = plsc.VectorSubcoreMesh(core_axis_name="c", subcore_axis_name="s")

    def body(*refs):
        idx = 5 + residual
        src_h, dst_h, p_h, q_h, r_h = refs[:5]
        prev_h = refs[5] if residual else None
        e_out = refs[idx]
        agg_out = refs[idx + 1] if with_agg else None
        sidx, didx, p_b, q_b, r_b, o_b, p_s, q_s = refs[
            idx + 1 + with_agg:idx + 9 + with_agg]
        off = idx + 9 + with_agg
        if residual:
            prev_b = refs[off]
            off += 1
        if with_agg:
            e_b, agg_s, z_b = refs[off], refs[off + 1], refs[off + 2]

        c = lax.axis_index("c")
        s = lax.axis_index("s")
        w = s * nc + c

        @pl.when(s == 0)
        def _():
            pltpu.sync_copy(p_h, p_s)

        @pl.when(s == 1)
        def _():
            pltpu.sync_copy(q_h, q_s)

        if with_agg:
            def zrow(i, carry):
                z_b[i] = jnp.zeros((de,), jnp.float32)
                return carry
            lax.fori_loop(0, rows, zrow, 0)
            pltpu.sync_copy(z_b, agg_s.at[pl.ds(s * rows, rows)])
        plsc.subcore_barrier()

        def chunk_body(j, carry):
            chunk = w + j * nw

            @pl.when(chunk < chunks)
            def _():
                base = chunk * pk
                pltpu.sync_copy(src_h.at[chunk], sidx)
                pltpu.sync_copy(dst_h.at[chunk], didx)
                pltpu.sync_copy(r_h.at[pl.ds(base, pk)], r_b)
                if residual:
                    pltpu.sync_copy(prev_h.at[pl.ds(base, pk)], prev_b)
                for k in range(nsub):
                    pltpu.sync_copy(p_s.at[sidx.at[k]],
                                    p_b.at[pl.ds(k * _SUB, _SUB)])
                    pltpu.sync_copy(q_s.at[didx.at[k]],
                                    q_b.at[pl.ds(k * _SUB, _SUB)])

                def row(i, carry):
                    for jj in range(8):
                        lo = pl.ds(jj * de, de)
                        e = jnp.maximum(
                            p_b[8 * i + jj] + q_b[8 * i + jj] + r_b[i, lo],
                            0.0)
                        if with_agg:
                            e_b[8 * i + jj] = e
                        if residual:
                            o_b[i, lo] = prev_b[i, lo] + e
                        else:
                            o_b[i, lo] = e
                    return carry
                lax.fori_loop(0, pk, row, 0)

                if with_agg:
                    for k in range(nsub):
                        pltpu.sync_copy(e_b.at[pl.ds(k * _SUB, _SUB)],
                                        agg_s.at[didx.at[k]], add=True)
                pltpu.sync_copy(o_b, e_out.at[pl.ds(base, pk)])
            return carry

        lax.fori_loop(0, maxj, chunk_body, 0)

        if with_agg:
            plsc.subcore_barrier()
            pltpu.sync_copy(agg_s.at[pl.ds(s * rows, rows)],
                            agg_out.at[pl.ds(c * n_pad + s * rows, rows)])

    fn = pl.kernel(
        body,
        out_type=out_type,
        scratch_types=scratch,
        mesh=mesh,
        compiler_params=pltpu.CompilerParams(use_tc_tiling_on_sc=False),
    )
    args = (src, dst, p, q, r) + ((prev,) if residual else ())
    outs = fn(*args)
    if with_agg:
        return outs[0], outs[1]
    return outs[0], None


def kernel(edge_index, x, z, We0, be0, Wn0, bn0, We1, be1, Wn1, bn1,
           We2, be2, Wn2, bn2):
    n, d = x.shape
    de = z.shape[1]
    n_edges = edge_index.shape[1]
    src = edge_index[0].reshape(n_edges // _CK, _CK // _SUB, _SUB)
    dst = edge_index[1].reshape(n_edges // _CK, _CK // _SUB, _SUB)
    x = x.astype(jnp.float32)
    z_p = z.reshape(n_edges // 8, 8 * de)
    n_pad = ((n + 15) // 16 + 7) // 8 * 8 * 16

    p0, q0 = _pq(x, We0[:d], We0[d:2 * d])
    r0 = _r(z_p, We0[2 * d:], be0)
    e0, agg0 = _edge_sc_call(src, dst, p0, q0, r0, prev=None, with_agg=True)
    x1, p1, q1 = _node(x, agg0[:n], agg0[n_pad:n_pad + n], Wn0[:d], Wn0[d:],
                       bn0, We1[:d], We1[d:2 * d], residual=False)

    r1 = _r(e0, We1[2 * d:], be1)
    e1, agg1 = _edge_sc_call(src, dst, p1, q1, r1, prev=e0, with_agg=True)
    x2, p2, q2 = _node(x1, agg1[:n], agg1[n_pad:n_pad + n], Wn1[:d], Wn1[d:],
                       bn1, We2[:d], We2[d:2 * d], residual=True)

    r2 = _r(e1, We2[2 * d:], be2)
    e2, _ = _edge_sc_call(src, dst, p2, q2, r2, prev=None, with_agg=False)
    return e2.reshape(n_edges, de)


# speedup vs baseline: 2.2458x; 1.0223x over previous
import functools

import jax
import jax.numpy as jnp
from jax import lax
from jax.experimental import pallas as pl
from jax.experimental.pallas import tpu as pltpu
from jax.experimental.pallas import tpu_sc as plsc


def _pq_body(x_ref, ws_ref, wd_ref, p_ref, q_ref):
    xb = x_ref[...]
    p_ref[...] = jnp.dot(xb, ws_ref[...], preferred_element_type=jnp.float32)
    q_ref[...] = jnp.dot(xb, wd_ref[...], preferred_element_type=jnp.float32)


def _pq(x, ws, wd):
    n, d = x.shape
    de = ws.shape[1]
    bn = 2000
    grid = (n // bn,)
    return pl.pallas_call(
        _pq_body,
        grid=grid,
        in_specs=[
            pl.BlockSpec((bn, d), lambda i: (i, 0)),
            pl.BlockSpec((d, de), lambda i: (0, 0)),
            pl.BlockSpec((d, de), lambda i: (0, 0)),
        ],
        out_specs=[
            pl.BlockSpec((bn, de), lambda i: (i, 0)),
            pl.BlockSpec((bn, de), lambda i: (i, 0)),
        ],
        out_shape=[
            jax.ShapeDtypeStruct((n, de), jnp.float32),
            jax.ShapeDtypeStruct((n, de), jnp.float32),
        ],
    )(x, ws, wd)


def _r_body(e_ref, w_ref, b_ref, r_ref):
    r_ref[...] = (
        jnp.dot(e_ref[...], w_ref[...], preferred_element_type=jnp.float32)
        + b_ref[...]
    )


def _r(e_p, w, b):
    ne8, lanes = e_p.shape
    de = w.shape[0]
    w_bd = jnp.kron(jnp.eye(lanes // de, dtype=jnp.float32), w)
    b_t = jnp.tile(b, lanes // de).reshape(1, lanes)
    be = 2000
    grid = (ne8 // be,)
    return pl.pallas_call(
        _r_body,
        grid=grid,
        in_specs=[
            pl.BlockSpec((be, lanes), lambda i: (i, 0)),
            pl.BlockSpec((lanes, lanes), lambda i: (0, 0)),
            pl.BlockSpec((1, lanes), lambda i: (0, 0)),
        ],
        out_specs=pl.BlockSpec((be, lanes), lambda i: (i, 0)),
        out_shape=jax.ShapeDtypeStruct((ne8, lanes), jnp.float32),
    )(e_p, w_bd, b_t)


def _node_body(residual, x_ref, a0_ref, a1_ref, wx_ref, wa_ref, b_ref,
               ws_ref, wd_ref, x_out, p_out, q_out):
    xb = x_ref[...]
    agg = a0_ref[...] + a1_ref[...]
    h = (
        jnp.dot(xb, wx_ref[...], preferred_element_type=jnp.float32)
        + jnp.dot(agg, wa_ref[...], preferred_element_type=jnp.float32)
        + b_ref[...]
    )
    h = jnp.maximum(h, 0.0)
    if residual:
        h = h + xb
    x_out[...] = h
    p_out[...] = jnp.dot(h, ws_ref[...], preferred_element_type=jnp.float32)
    q_out[...] = jnp.dot(h, wd_ref[...], preferred_element_type=jnp.float32)


def _node(x, a0, a1, wx, wa, b, ws_next, wd_next, residual):
    n, d = x.shape
    de = a0.shape[1]
    bn = 2000
    grid = (n // bn,)
    return pl.pallas_call(
        functools.partial(_node_body, residual),
        grid=grid,
        in_specs=[
            pl.BlockSpec((bn, d), lambda i: (i, 0)),
            pl.BlockSpec((bn, de), lambda i: (i, 0)),
            pl.BlockSpec((bn, de), lambda i: (i, 0)),
            pl.BlockSpec((d, d), lambda i: (0, 0)),
            pl.BlockSpec((de, d), lambda i: (0, 0)),
            pl.BlockSpec((1, d), lambda i: (0, 0)),
            pl.BlockSpec((d, de), lambda i: (0, 0)),
            pl.BlockSpec((d, de), lambda i: (0, 0)),
        ],
        out_specs=[
            pl.BlockSpec((bn, d), lambda i: (i, 0)),
            pl.BlockSpec((bn, de), lambda i: (i, 0)),
            pl.BlockSpec((bn, de), lambda i: (i, 0)),
        ],
        out_shape=[
            jax.ShapeDtypeStruct((n, d), jnp.float32),
            jax.ShapeDtypeStruct((n, de), jnp.float32),
            jax.ShapeDtypeStruct((n, de), jnp.float32),
        ],
    )(x, a0, a1, wx, wa, b.reshape(1, d), ws_next, wd_next)


_CK = 512
_SUB = 128


def _edge_sc_call(src, dst, p, q, r, prev, with_agg):
    chunks, nsub, sub = src.shape
    e_edges = chunks * nsub * sub
    n, de = p.shape
    pk = _CK // 8
    residual = prev is not None

    try:
        info = plsc.get_sparse_core_info()
        nc, ns = int(info.num_cores), int(info.num_subcores)
    except Exception:
        nc, ns = 2, 16
    nw = nc * ns

    maxj = (chunks + nw - 1) // nw
    rows = ((n + ns - 1) // ns + 7) // 8 * 8
    n_pad = rows * ns

    out_type = [jax.ShapeDtypeStruct((e_edges // 8, 8 * de), jnp.float32)]
    if with_agg:
        out_type.append(jax.ShapeDtypeStruct((nc * n_pad, de), jnp.float32))

    scratch = [
        pltpu.VMEM((nsub, _SUB), jnp.int32),
        pltpu.VMEM((nsub, _SUB), jnp.int32),
        pltpu.VMEM((_CK, de), jnp.float32),
        pltpu.VMEM((pk, 8 * de), jnp.float32),
        pltpu.VMEM((pk, 8 * de), jnp.float32),
        pltpu.VMEM_SHARED((n, de), jnp.float32),
        pltpu.VMEM_SHARED((n, de), jnp.float32),
    ]
    if residual:
        scratch.append(pltpu.VMEM((pk, 8 * de), jnp.float32))
    if with_agg:
        scratch.append(pltpu.VMEM((_CK, de), jnp.float32))
        scratch.append(pltpu.VMEM_SHARED((n_pad, de), jnp.float32))
        scratch.append(pltpu.VMEM((rows, de), jnp.float32))

    mesh = plsc.VectorSubcoreMesh(core_axis_name="c", subcore_axis_name="s")

    def body(*refs):
        idx = 5 + residual
        src_h, dst_h, p_h, q_h, r_h = refs[:5]
        prev_h = refs[5] if residual else None
        e_out = refs[idx]
        agg_out = refs[idx + 1] if with_agg else None
        sidx, didx, pq_b, r_b, o_b, p_s, q_s = refs[
            idx + 1 + with_agg:idx + 8 + with_agg]
        off = idx + 8 + with_agg
        if residual:
            prev_b = refs[off]
            off += 1
        if with_agg:
            e_b, agg_s, z_b = refs[off], refs[off + 1], refs[off + 2]

        c = lax.axis_index("c")
        s = lax.axis_index("s")
        w = s * nc + c

        @pl.when(s == 0)
        def _():
            pltpu.sync_copy(p_h, p_s)

        @pl.when(s == 1)
        def _():
            pltpu.sync_copy(q_h, q_s)

        if with_agg:
            def zrow(i, carry):
                z_b[i] = jnp.zeros((de,), jnp.float32)
                return carry
            lax.fori_loop(0, rows, zrow, 0)
            pltpu.sync_copy(z_b, agg_s.at[pl.ds(s * rows, rows)])
        plsc.subcore_barrier()

        def chunk_body(j, carry):
            chunk = w + j * nw

            @pl.when(chunk < chunks)
            def _():
                base = chunk * pk
                pltpu.sync_copy(src_h.at[chunk], sidx)
                pltpu.sync_copy(dst_h.at[chunk], didx)
                for k in range(nsub):
                    pltpu.sync_copy(p_s.at[sidx.at[k]],
                                    pq_b.at[pl.ds(k * _SUB, _SUB)])
                for k in range(nsub):
                    pltpu.sync_copy(q_s.at[didx.at[k]],
                                    pq_b.at[pl.ds(k * _SUB, _SUB)], add=True)
                pltpu.sync_copy(r_h.at[pl.ds(base, pk)], r_b)
                if residual:
                    pltpu.sync_copy(prev_h.at[pl.ds(base, pk)], prev_b)

                def row(i, carry):
                    for jj in range(8):
                        lo = pl.ds(jj * de, de)
                        e = jnp.maximum(pq_b[8 * i + jj] + r_b[i, lo], 0.0)
                        if with_agg:
                            e_b[8 * i + jj] = e
                        if residual:
                            o_b[i, lo] = prev_b[i, lo] + e
                        else:
                            o_b[i, lo] = e
                    return carry
                lax.fori_loop(0, pk, row, 0)

                if with_agg:
                    for k in range(nsub):
                        pltpu.sync_copy(e_b.at[pl.ds(k * _SUB, _SUB)],
                                        agg_s.at[didx.at[k]], add=True)
                pltpu.sync_copy(o_b, e_out.at[pl.ds(base, pk)])
            return carry

        lax.fori_loop(0, maxj, chunk_body, 0)

        if with_agg:
            plsc.subcore_barrier()
            pltpu.sync_copy(agg_s.at[pl.ds(s * rows, rows)],
                            agg_out.at[pl.ds(c * n_pad + s * rows, rows)])

    fn = pl.kernel(
        body,
        out_type=out_type,
        scratch_types=scratch,
        mesh=mesh,
        compiler_params=pltpu.CompilerParams(use_tc_tiling_on_sc=False),
    )
    args = (src, dst, p, q, r) + ((prev,) if residual else ())
    outs = fn(*args)
    if with_agg:
        return outs[0], outs[1]
    return outs[0], None


def kernel(edge_index, x, z, We0, be0, Wn0, bn0, We1, be1, Wn1, bn1,
           We2, be2, Wn2, bn2):
    n, d = x.shape
    de = z.shape[1]
    n_edges = edge_index.shape[1]
    src = edge_index[0].reshape(n_edges // _CK, _CK // _SUB, _SUB)
    dst = edge_index[1].reshape(n_edges // _CK, _CK // _SUB, _SUB)
    x = x.astype(jnp.float32)
    z_p = z.reshape(n_edges // 8, 8 * de)
    n_pad = ((n + 15) // 16 + 7) // 8 * 8 * 16

    p0, q0 = _pq(x, We0[:d], We0[d:2 * d])
    r0 = _r(z_p, We0[2 * d:], be0)
    e0, agg0 = _edge_sc_call(src, dst, p0, q0, r0, prev=None, with_agg=True)
    x1, p1, q1 = _node(x, agg0[:n], agg0[n_pad:n_pad + n], Wn0[:d], Wn0[d:],
                       bn0, We1[:d], We1[d:2 * d], residual=False)

    r1 = _r(e0, We1[2 * d:], be1)
    e1, agg1 = _edge_sc_call(src, dst, p1, q1, r1, prev=e0, with_agg=True)
    x2, p2, q2 = _node(x1, agg1[:n], agg1[n_pad:n_pad + n], Wn1[:d], Wn1[d:],
                       bn1, We2[:d], We2[d:2 * d], residual=True)

    r2 = _r(e1, We2[2 * d:], be2)
    e2, _ = _edge_sc_call(src, dst, p2, q2, r2, prev=None, with_agg=False)
    return e2.reshape(n_edges, de)
